# Initial kernel scaffold; baseline (speedup 1.0000x reference)
import jax
import jax.numpy as jnp
from jax.experimental import pallas as pl


def kernel(inputs, hash_table):
    raise NotImplementedError("write your pallas kernel here")


# speedup vs baseline: 217.4280x; 217.4280x over previous
import functools
import math

import numpy as np

import jax
import jax.numpy as jnp
from jax import lax
from jax.experimental import pallas as pl
from jax.experimental.pallas import tpu as pltpu
from jax.experimental.pallas import tpu_sc as plsc

L = 16
T = 2 ** 14
F = 2
N_MIN = 16
N_MAX = 512
BOUND = 3.0
_b = math.exp((math.log(N_MAX) - math.log(N_MIN)) / (L - 1))
NS = [int(N_MIN * _b ** i) for i in range(L)]
WIDTHS = [np.float32(1.0 / n) for n in NS]
P1_I32 = np.int32(2654435761 - 2 ** 32)
P2_I32 = np.int32(805459861)
MASK = T - 1

BATCH = 262144
NCORES = 2
NSUB = 16
HALF = BATCH // NCORES
CHUNK = 8192
NCHUNK = HALF // CHUNK
NGRP = CHUNK // 16
TBL = T * F


def _width_for_level(l):
    w = jnp.float32(WIDTHS[0])
    for i in range(1, L):
        w = jnp.where(l == np.int32(i), jnp.float32(WIDTHS[i]), w)
    return w


def _sc_body(x_hbm, y_hbm, z_hbm, ht_hbm, out_hbm, table_v, xb, yb, zb, o0, o1,
             sem_in, sem_out):
    lvl = lax.axis_index("s")
    core = lax.axis_index("c")
    pltpu.async_copy(ht_hbm.at[pl.ds(lvl * np.int32(TBL), TBL)], table_v,
                     sem_in).wait()

    w = _width_for_level(lvl)
    wv = lax.broadcast(w, (16,))

    zero_i = core * np.int32(0)
    ngrp_i = zero_i + np.int32(NGRP)
    one_i = zero_i + np.int32(1)

    for k in range(NCHUNK):
        base = core * np.int32(HALF) + np.int32(k * CHUNK)
        cx = pltpu.async_copy(x_hbm.at[pl.ds(base, CHUNK)], xb, sem_in)
        cy = pltpu.async_copy(y_hbm.at[pl.ds(base, CHUNK)], yb, sem_in)
        cz = pltpu.async_copy(z_hbm.at[pl.ds(base, CHUNK)], zb, sem_in)
        cx.wait()
        cy.wait()
        cz.wait()

        @plsc.parallel_loop(zero_i, ngrp_i, one_i, carry=jnp.int32(0))
        def group_body(g, off):
            xv = xb[pl.ds(off, 16)]
            yv = yb[pl.ds(off, 16)]
            zv = zb[pl.ds(off, 16)]
            x = (xv + 3.0) / 6.0
            y = (yv + 3.0) / 6.0
            z = (zv + 3.0) / 6.0
            xi = (x / wv).astype(jnp.int32)
            yi = (y / wv).astype(jnp.int32)
            zi = (z / wv).astype(jnp.int32)
            xp = (x - xi.astype(jnp.float32) * wv) / wv
            yp = (y - yi.astype(jnp.float32) * wv) / wv
            zp = (z - zi.astype(jnp.float32) * wv) / wv

            hx0 = xi
            hx1 = xi + np.int32(1)
            hy0 = yi * P1_I32
            hy1 = hy0 + P1_I32
            hz0 = zi * P2_I32
            hz1 = hz0 + P2_I32

            u = 1.0 - xp
            v = 1.0 - yp
            t = 1.0 - zp
            uv = u * v
            uy = u * yp
            xv2 = xp * v
            xy = xp * yp
            cs = (uv * t, uv * zp, uy * t, uy * zp,
                  xv2 * t, xv2 * zp, xy * t, xy * zp)
            hs = (hx0 ^ hy0 ^ hz0, hx0 ^ hy0 ^ hz1,
                  hx0 ^ hy1 ^ hz0, hx0 ^ hy1 ^ hz1,
                  hx1 ^ hy0 ^ hz0, hx1 ^ hy0 ^ hz1,
                  hx1 ^ hy1 ^ hz0, hx1 ^ hy1 ^ hz1)
            acc0 = None
            acc1 = None
            for h, cw in zip(hs, cs):
                i0 = (h & np.int32(MASK)) << np.int32(1)
                fa = plsc.load_gather(table_v, [i0])
                fb = plsc.load_gather(table_v, [i0 + np.int32(1)])
                if acc0 is None:
                    acc0 = fa * cw
                    acc1 = fb * cw
                else:
                    acc0 = acc0 + fa * cw
                    acc1 = acc1 + fb * cw
            o0[pl.ds(off, 16)] = acc0
            o1[pl.ds(off, 16)] = acc1
            return off + np.int32(16)

        obase = lvl * np.int32(2 * BATCH) + base
        c0 = pltpu.async_copy(o0, out_hbm.at[pl.ds(obase, CHUNK)], sem_out)
        c1 = pltpu.async_copy(o1, out_hbm.at[pl.ds(obase + np.int32(BATCH), CHUNK)], sem_out)
        c0.wait()
        c1.wait()


@functools.partial(jax.jit, static_argnums=())
def kernel(inputs, hash_table):
    x = inputs[:, 0]
    y = inputs[:, 1]
    z = inputs[:, 2]
    ht_flat = hash_table.reshape(L * T * F)
    mesh = plsc.VectorSubcoreMesh(core_axis_name="c", subcore_axis_name="s")
    sc_fn = pl.kernel(
        _sc_body,
        out_type=jax.ShapeDtypeStruct((L * F * BATCH,), jnp.float32),
        mesh=mesh,
        compiler_params=pltpu.CompilerParams(needs_layout_passes=False),
        scratch_types=[
            pltpu.VMEM((TBL,), jnp.float32),
            pltpu.VMEM((CHUNK,), jnp.float32),
            pltpu.VMEM((CHUNK,), jnp.float32),
            pltpu.VMEM((CHUNK,), jnp.float32),
            pltpu.VMEM((CHUNK,), jnp.float32),
            pltpu.VMEM((CHUNK,), jnp.float32),
            pltpu.SemaphoreType.DMA,
            pltpu.SemaphoreType.DMA,
        ],
    )
    out = sc_fn(x, y, z, ht_flat)
    out3 = out.reshape(L, F, BATCH)
    return jnp.transpose(out3, (2, 0, 1)).reshape(BATCH, L * F)
